# Initial kernel scaffold; baseline (speedup 1.0000x reference)
import jax
import jax.numpy as jnp
from jax.experimental import pallas as pl


def kernel(points_features, points_neighbor, p1f, p1w, p1u, p2f, p2w, p2u, p3f, p3w, p3u):
    raise NotImplementedError("write your pallas kernel here")


# speedup vs baseline: 1.4961x; 1.4961x over previous
import functools

import jax
import jax.numpy as jnp
from jax import lax
from jax.experimental import pallas as pl
from jax.experimental.pallas import tpu as pltpu
from jax.experimental.pallas import tpu_sc as plsc

EPS_BN = 1e-5
BLK = 1000
NC, NS = 2, 16
NW = NC * NS


def _fold_pn(params):
    W1, b1, g1, be1, W2, b2, g2, be2 = params
    s = 1.0 / jnp.sqrt(1.0 + EPS_BN)
    A1 = W1.T * (g1 * s)[None, :]
    c1 = (b1 * (g1 * s) + be1).reshape(1, -1)
    A2 = W2.T * (g2 * s)[None, :]
    c2 = (b2 * (g2 * s) + be2).reshape(1, -1)
    return A1, c1, A2, c2


def _pn2(x, a1, c1, a2, c2):
    h = jnp.maximum(jnp.dot(x, a1, preferred_element_type=jnp.float32) + c1, 0.0)
    return jnp.dot(h, a2, preferred_element_type=jnp.float32) + c2


def _colsumsq_kernel(x_ref, o_ref):
    p = jnp.sum(x_ref[...] * x_ref[...], axis=0, keepdims=True)

    @pl.when(pl.program_id(0) == 0)
    def _():
        o_ref[...] = p

    @pl.when(pl.program_id(0) > 0)
    def _():
        o_ref[...] += p


def _prep_kernel(ss_ref, x_ref, pn_ref, a1_ref, c1_ref, a2_ref, c2_ref,
                 t1_ref, nb_ref, pf6_ref):
    x = x_ref[...]
    inv = 1.0 / jnp.maximum(jnp.sqrt(ss_ref[0:1, 0:3]), 1e-12)
    pf6 = jnp.concatenate([x[:, 0:3] * inv, x[:, 3:6] * (1.0 / 255.0)], axis=1)
    f1 = _pn2(pf6, a1_ref[...], c1_ref[...], a2_ref[...], c2_ref[...])
    sel = jnp.concatenate([x[:, 0:3], x[:, 6:9]], axis=1)
    t1_ref[...] = jnp.concatenate(
        [sel, f1, jnp.zeros((BLK, 14), jnp.float32)], axis=1)
    pf6_ref[...] = pf6
    row = pl.program_id(0) * BLK + lax.broadcasted_iota(jnp.int32, (BLK, 9), 0)
    pnv = pn_ref[...]
    nb_ref[...] = jnp.where(pnv == 0, row, pnv)


def _make_gather(n_rows, d, b_pad):
    bpw = b_pad // NW
    ch = 38400 // d
    n_ch = bpw // ch
    assert bpw % ch == 0 and ch % 8 == 0 and bpw % 8 == 0
    mesh = plsc.VectorSubcoreMesh(core_axis_name="c", subcore_axis_name="s")

    @functools.partial(
        pl.kernel,
        mesh=mesh,
        out_type=jax.ShapeDtypeStruct((b_pad, d), jnp.float32),
        scratch_types=[
            pltpu.VMEM((bpw,), jnp.int32),
            pltpu.VMEM((2, ch, d), jnp.float32),
            pltpu.SemaphoreType.DMA,
            pltpu.SemaphoreType.DMA,
        ],
        compiler_params=pltpu.CompilerParams(use_tc_tiling_on_sc=False),
    )
    def gather(table_hbm, idx_hbm, out_hbm, idx_v, bufs, gsem, wsem):
        wid = lax.axis_index("s") * NC + lax.axis_index("c")
        base = wid * bpw
        pltpu.sync_copy(idx_hbm.at[pl.ds(base, bpw)], idx_v)

        def body(g, _):
            off = g * ch
            pltpu.async_copy(
                table_hbm.at[idx_v.at[pl.ds(off, ch)]], bufs.at[0], gsem
            ).wait()
            pltpu.async_copy(
                bufs.at[0], out_hbm.at[pl.ds(base + off, ch)], wsem
            ).wait()
            return 0

        lax.fori_loop(0, n_ch, body, 0)

    return gather


def _s1_kernel(e_ref, t1_ref,
               wa1, wc1, wa2, wc2, ua1, uc1, ua2, uc2, fa1, fc1, fa2, fc2,
               feas_ref, t2_ref, xyz_ref):
    e = e_ref[...]
    sel = t1_ref[:, 0:6]
    xs = [e[:, 32 * j: 32 * j + 6] - sel for j in range(9)]
    x_all = jnp.concatenate(xs, axis=0)
    g_all = jnp.concatenate(
        [e[:, 32 * j + 6: 32 * j + 18] for j in range(9)], axis=0)
    w = _pn2(x_all, wa1[...], wc1[...], wa2[...], wc2[...])
    p = g_all * w
    z = jnp.concatenate(
        [p[j * BLK:(j + 1) * BLK, :] for j in range(9)], axis=1)
    feas = _pn2(z, ua1[...], uc1[...], ua2[...], uc2[...])
    f2 = _pn2(feas, fa1[...], fc1[...], fa2[...], fc2[...])
    feas_ref[...] = feas
    t2_ref[...] = jnp.concatenate([f2, jnp.zeros((BLK, 8), jnp.float32)], axis=1)
    zero2 = jnp.zeros((BLK, 2), jnp.float32)
    xyz_ref[...] = jnp.concatenate(
        sum(([xj, zero2] for xj in xs), []), axis=1)


def _s2_kernel(e_ref, xyz_ref,
               wa1, wc1, wa2, wc2, ua1, uc1, ua2, uc2, fa1, fc1, fa2, fc2,
               feas_ref, t3_ref):
    e = e_ref[...]
    xyz = xyz_ref[...]
    x_all = jnp.concatenate(
        [xyz[:, 8 * j: 8 * j + 6] for j in range(9)], axis=0)
    g_all = jnp.concatenate(
        [e[:, 32 * j: 32 * j + 24] for j in range(9)], axis=0)
    w = _pn2(x_all, wa1[...], wc1[...], wa2[...], wc2[...])
    p = g_all * w
    z = jnp.concatenate(
        [p[j * BLK:(j + 1) * BLK, :] for j in range(9)], axis=1)
    feas = _pn2(z, ua1[...], uc1[...], ua2[...], uc2[...])
    f3 = _pn2(feas, fa1[...], fc1[...], fa2[...], fc2[...])
    feas_ref[...] = feas
    t3_ref[...] = f3


def _s3_kernel(e_ref, xyz_ref,
               wa1, wc1, wa2, wc2, ua1, uc1, ua2, uc2,
               feas_ref):
    e = e_ref[...]
    xyz = xyz_ref[...]
    x_all = jnp.concatenate(
        [xyz[:, 8 * j: 8 * j + 6] for j in range(9)], axis=0)
    g_all = jnp.concatenate(
        [e[:, 48 * j: 48 * j + 48] for j in range(9)], axis=0)
    w = _pn2(x_all, wa1[...], wc1[...], wa2[...], wc2[...])
    p = g_all * w
    z = jnp.concatenate(
        [p[j * BLK:(j + 1) * BLK, :] for j in range(9)], axis=1)
    feas_ref[...] = _pn2(z, ua1[...], uc1[...], ua2[...], uc2[...])


def _full_spec(shape):
    return pl.BlockSpec(shape, lambda i: (0, 0))


def kernel(points_features, points_neighbor, p1f, p1w, p1u, p2f, p2w, p2u,
           p3f, p3w, p3u):
    n, m = points_features.shape[0], points_neighbor.shape[1]
    b = n * m
    b_pad = ((b + 8 * NW - 1) // (8 * NW)) * (8 * NW)
    while ((b_pad // NW) % (38400 // 48) or (b_pad // NW) % (38400 // 32)
           or b_pad % (9 * BLK)):
        b_pad += 8 * NW
    grid = (n // BLK,)

    f1p = _fold_pn(p1f)
    w1p, u1p, f2p = _fold_pn(p1w), _fold_pn(p1u), _fold_pn(p2f)
    w2p, u2p, f3p = _fold_pn(p2w), _fold_pn(p2u), _fold_pn(p3f)
    w3p, u3p = _fold_pn(p3w), _fold_pn(p3u)

    ss = pl.pallas_call(
        _colsumsq_kernel,
        grid=grid,
        in_specs=[pl.BlockSpec((BLK, 9), lambda i: (i, 0))],
        out_specs=pl.BlockSpec((1, 9), lambda i: (0, 0)),
        out_shape=jax.ShapeDtypeStruct((1, 9), jnp.float32),
    )(points_features)

    wspecs1 = [_full_spec(x.shape) for x in f1p]
    t1, nb, pf6 = pl.pallas_call(
        _prep_kernel,
        grid=grid,
        in_specs=[_full_spec((1, 9)),
                  pl.BlockSpec((BLK, 9), lambda i: (i, 0)),
                  pl.BlockSpec((BLK, 9), lambda i: (i, 0))] + wspecs1,
        out_specs=[pl.BlockSpec((BLK, 32), lambda i: (i, 0)),
                   pl.BlockSpec((BLK, 9), lambda i: (i, 0)),
                   pl.BlockSpec((BLK, 6), lambda i: (i, 0))],
        out_shape=[jax.ShapeDtypeStruct((n, 32), jnp.float32),
                   jax.ShapeDtypeStruct((n, 9), jnp.int32),
                   jax.ShapeDtypeStruct((n, 6), jnp.float32)],
    )(ss, points_features, points_neighbor, *f1p)

    idx = jnp.pad(nb.reshape(-1), (0, b_pad - b))

    gather32 = _make_gather(n, 32, b_pad)
    gather48 = _make_gather(n, 48, b_pad)

    e1 = gather32(t1, idx).reshape(b_pad // 9, 9 * 32)

    wspecs = [_full_spec(x.shape) for x in (*w1p, *u1p, *f2p)]
    feas1, t2, xyz = pl.pallas_call(
        _s1_kernel,
        grid=grid,
        in_specs=[pl.BlockSpec((BLK, 288), lambda i: (i, 0)),
                  pl.BlockSpec((BLK, 32), lambda i: (i, 0))] + wspecs,
        out_specs=[pl.BlockSpec((BLK, 12), lambda i: (i, 0)),
                   pl.BlockSpec((BLK, 32), lambda i: (i, 0)),
                   pl.BlockSpec((BLK, 72), lambda i: (i, 0))],
        out_shape=[jax.ShapeDtypeStruct((n, 12), jnp.float32),
                   jax.ShapeDtypeStruct((n, 32), jnp.float32),
                   jax.ShapeDtypeStruct((n, 72), jnp.float32)],
    )(e1, t1, *w1p, *u1p, *f2p)

    e2 = gather32(t2, idx).reshape(b_pad // 9, 9 * 32)

    wspecs = [_full_spec(x.shape) for x in (*w2p, *u2p, *f3p)]
    feas2, t3 = pl.pallas_call(
        _s2_kernel,
        grid=grid,
        in_specs=[pl.BlockSpec((BLK, 288), lambda i: (i, 0)),
                  pl.BlockSpec((BLK, 72), lambda i: (i, 0))] + wspecs,
        out_specs=[pl.BlockSpec((BLK, 24), lambda i: (i, 0)),
                   pl.BlockSpec((BLK, 48), lambda i: (i, 0))],
        out_shape=[jax.ShapeDtypeStruct((n, 24), jnp.float32),
                   jax.ShapeDtypeStruct((n, 48), jnp.float32)],
    )(e2, xyz, *w2p, *u2p, *f3p)

    e3 = gather48(t3, idx).reshape(b_pad // 9, 9 * 48)

    wspecs = [_full_spec(x.shape) for x in (*w3p, *u3p)]
    feas3 = pl.pallas_call(
        _s3_kernel,
        grid=grid,
        in_specs=[pl.BlockSpec((BLK, 432), lambda i: (i, 0)),
                  pl.BlockSpec((BLK, 72), lambda i: (i, 0))] + wspecs,
        out_specs=pl.BlockSpec((BLK, 48), lambda i: (i, 0)),
        out_shape=jax.ShapeDtypeStruct((n, 48), jnp.float32),
    )(e3, xyz, *w3p, *u3p)

    return jnp.concatenate([feas3, feas2, feas1, pf6], axis=1)


# speedup vs baseline: 4.8377x; 3.2335x over previous
import functools

import jax
import jax.numpy as jnp
from jax import lax
from jax.experimental import pallas as pl
from jax.experimental.pallas import tpu as pltpu
from jax.experimental.pallas import tpu_sc as plsc

EPS_BN = 1e-5
BLK = 1000
NC, NS = 2, 16
NW = NC * NS


def _fold_pn(params):
    W1, b1, g1, be1, W2, b2, g2, be2 = params
    s = 1.0 / jnp.sqrt(1.0 + EPS_BN)
    A1 = W1.T * (g1 * s)[None, :]
    c1 = (b1 * (g1 * s) + be1).reshape(1, -1)
    A2 = W2.T * (g2 * s)[None, :]
    c2 = (b2 * (g2 * s) + be2).reshape(1, -1)
    return A1, c1, A2, c2


def _pn2(x, a1, c1, a2, c2):
    h = jnp.maximum(jnp.dot(x, a1, preferred_element_type=jnp.float32) + c1, 0.0)
    return jnp.dot(h, a2, preferred_element_type=jnp.float32) + c2


def _colsumsq_kernel(x_ref, o_ref):
    p = jnp.sum(x_ref[...] * x_ref[...], axis=0, keepdims=True)

    @pl.when(pl.program_id(0) == 0)
    def _():
        o_ref[...] = p

    @pl.when(pl.program_id(0) > 0)
    def _():
        o_ref[...] += p


def _prep_kernel(ss_ref, x_ref, pn_ref, a1_ref, c1_ref, a2_ref, c2_ref,
                 t1_ref, nb_ref, pf6_ref):
    x = x_ref[...]
    inv = 1.0 / jnp.maximum(jnp.sqrt(ss_ref[0:1, 0:3]), 1e-12)
    pf6 = jnp.concatenate([x[:, 0:3] * inv, x[:, 3:6] * (1.0 / 255.0)], axis=1)
    f1 = _pn2(pf6, a1_ref[...], c1_ref[...], a2_ref[...], c2_ref[...])
    sel = jnp.concatenate([x[:, 0:3], x[:, 6:9]], axis=1)
    t1_ref[...] = jnp.concatenate(
        [sel, f1, jnp.zeros((BLK, 14), jnp.float32)], axis=1)
    pf6_ref[...] = pf6
    row = pl.program_id(0) * BLK + lax.broadcasted_iota(jnp.int32, (BLK, 9), 0)
    pnv = pn_ref[...]
    nb_ref[...] = jnp.where(pnv == 0, row, pnv)


def _make_gather(n_rows, d, b_pad):
    bpw = b_pad // NW
    ch = 46080 // d
    n_ch = bpw // ch
    assert bpw % ch == 0 and ch % 8 == 0 and n_ch % 2 == 0
    mesh = plsc.VectorSubcoreMesh(core_axis_name="c", subcore_axis_name="s")

    @functools.partial(
        pl.kernel,
        mesh=mesh,
        out_type=jax.ShapeDtypeStruct((b_pad, d), jnp.float32),
        scratch_types=[
            pltpu.VMEM((bpw,), jnp.int32),
            pltpu.VMEM((2, ch, d), jnp.float32),
            pltpu.SemaphoreType.DMA,
            pltpu.SemaphoreType.DMA,
            pltpu.SemaphoreType.DMA,
            pltpu.SemaphoreType.DMA,
        ],
        compiler_params=pltpu.CompilerParams(use_tc_tiling_on_sc=False),
    )
    def gather(table_hbm, idx_hbm, out_hbm, idx_v, bufs,
               gsem0, gsem1, wsem0, wsem1):
        wid = lax.axis_index("s") * NC + lax.axis_index("c")
        base = wid * bpw
        gsems = (gsem0, gsem1)
        wsems = (wsem0, wsem1)
        pltpu.sync_copy(idx_hbm.at[pl.ds(base, bpw)], idx_v)

        def start_g(g, b):
            return pltpu.async_copy(
                table_hbm.at[idx_v.at[pl.ds(g * ch, ch)]], bufs.at[b], gsems[b])

        def start_w(g, b):
            return pltpu.async_copy(
                bufs.at[b], out_hbm.at[pl.ds(base + g * ch, ch)], wsems[b])

        cps = [start_g(0, 0), start_g(1, 1)]
        wps = [None, None]
        for g in range(n_ch):
            b = g % 2
            cps[b].wait()
            wps[b] = start_w(g, b)
            if g + 2 < n_ch:
                wps[b].wait()
                cps[b] = start_g(g + 2, b)
            else:
                wps[b].wait()

    return gather


def _s1_kernel(e_ref, t1_ref,
               wa1, wc1, wa2, wc2, ua1, uc1, ua2, uc2, fa1, fc1, fa2, fc2,
               feas_ref, t2_ref, xyz_ref):
    e = e_ref[...]
    sel = t1_ref[:, 0:6]
    xs = [e[:, 32 * j: 32 * j + 6] - sel for j in range(9)]
    x_all = jnp.concatenate(xs, axis=0)
    g_all = jnp.concatenate(
        [e[:, 32 * j + 6: 32 * j + 18] for j in range(9)], axis=0)
    w = _pn2(x_all, wa1[...], wc1[...], wa2[...], wc2[...])
    p = g_all * w
    z = jnp.concatenate(
        [p[j * BLK:(j + 1) * BLK, :] for j in range(9)], axis=1)
    feas = _pn2(z, ua1[...], uc1[...], ua2[...], uc2[...])
    f2 = _pn2(feas, fa1[...], fc1[...], fa2[...], fc2[...])
    feas_ref[...] = feas
    t2_ref[...] = jnp.concatenate([f2, jnp.zeros((BLK, 8), jnp.float32)], axis=1)
    zero2 = jnp.zeros((BLK, 2), jnp.float32)
    xyz_ref[...] = jnp.concatenate(
        sum(([xj, zero2] for xj in xs), []), axis=1)


def _s2_kernel(e_ref, xyz_ref,
               wa1, wc1, wa2, wc2, ua1, uc1, ua2, uc2, fa1, fc1, fa2, fc2,
               feas_ref, t3_ref):
    e = e_ref[...]
    xyz = xyz_ref[...]
    x_all = jnp.concatenate(
        [xyz[:, 8 * j: 8 * j + 6] for j in range(9)], axis=0)
    g_all = jnp.concatenate(
        [e[:, 32 * j: 32 * j + 24] for j in range(9)], axis=0)
    w = _pn2(x_all, wa1[...], wc1[...], wa2[...], wc2[...])
    p = g_all * w
    z = jnp.concatenate(
        [p[j * BLK:(j + 1) * BLK, :] for j in range(9)], axis=1)
    feas = _pn2(z, ua1[...], uc1[...], ua2[...], uc2[...])
    f3 = _pn2(feas, fa1[...], fc1[...], fa2[...], fc2[...])
    feas_ref[...] = feas
    t3_ref[...] = f3


def _s3_kernel(e_ref, xyz_ref,
               wa1, wc1, wa2, wc2, ua1, uc1, ua2, uc2,
               feas_ref):
    e = e_ref[...]
    xyz = xyz_ref[...]
    x_all = jnp.concatenate(
        [xyz[:, 8 * j: 8 * j + 6] for j in range(9)], axis=0)
    g_all = jnp.concatenate(
        [e[:, 48 * j: 48 * j + 48] for j in range(9)], axis=0)
    w = _pn2(x_all, wa1[...], wc1[...], wa2[...], wc2[...])
    p = g_all * w
    z = jnp.concatenate(
        [p[j * BLK:(j + 1) * BLK, :] for j in range(9)], axis=1)
    feas_ref[...] = _pn2(z, ua1[...], uc1[...], ua2[...], uc2[...])


def _full_spec(shape):
    return pl.BlockSpec(shape, lambda i: (0, 0))


def kernel(points_features, points_neighbor, p1f, p1w, p1u, p2f, p2w, p2u,
           p3f, p3w, p3u):
    n, m = points_features.shape[0], points_neighbor.shape[1]
    b = n * m
    b_pad = ((b + 8 * NW - 1) // (8 * NW)) * (8 * NW)
    while ((b_pad // NW) % (2 * (46080 // 48)) or (b_pad // NW) % (2 * (46080 // 32))
           or b_pad % 9):
        b_pad += 8 * NW
    grid = (n // BLK,)

    f1p = _fold_pn(p1f)
    w1p, u1p, f2p = _fold_pn(p1w), _fold_pn(p1u), _fold_pn(p2f)
    w2p, u2p, f3p = _fold_pn(p2w), _fold_pn(p2u), _fold_pn(p3f)
    w3p, u3p = _fold_pn(p3w), _fold_pn(p3u)

    ss = pl.pallas_call(
        _colsumsq_kernel,
        grid=grid,
        in_specs=[pl.BlockSpec((BLK, 9), lambda i: (i, 0))],
        out_specs=pl.BlockSpec((1, 9), lambda i: (0, 0)),
        out_shape=jax.ShapeDtypeStruct((1, 9), jnp.float32),
    )(points_features)

    wspecs1 = [_full_spec(x.shape) for x in f1p]
    t1, nb, pf6 = pl.pallas_call(
        _prep_kernel,
        grid=grid,
        in_specs=[_full_spec((1, 9)),
                  pl.BlockSpec((BLK, 9), lambda i: (i, 0)),
                  pl.BlockSpec((BLK, 9), lambda i: (i, 0))] + wspecs1,
        out_specs=[pl.BlockSpec((BLK, 32), lambda i: (i, 0)),
                   pl.BlockSpec((BLK, 9), lambda i: (i, 0)),
                   pl.BlockSpec((BLK, 6), lambda i: (i, 0))],
        out_shape=[jax.ShapeDtypeStruct((n, 32), jnp.float32),
                   jax.ShapeDtypeStruct((n, 9), jnp.int32),
                   jax.ShapeDtypeStruct((n, 6), jnp.float32)],
    )(ss, points_features, points_neighbor, *f1p)

    idx = jnp.pad(nb.reshape(-1), (0, b_pad - b))

    gather32 = _make_gather(n, 32, b_pad)
    gather48 = _make_gather(n, 48, b_pad)

    e1 = gather32(t1, idx).reshape(b_pad // 9, 9 * 32)

    wspecs = [_full_spec(x.shape) for x in (*w1p, *u1p, *f2p)]
    feas1, t2, xyz = pl.pallas_call(
        _s1_kernel,
        grid=grid,
        in_specs=[pl.BlockSpec((BLK, 288), lambda i: (i, 0)),
                  pl.BlockSpec((BLK, 32), lambda i: (i, 0))] + wspecs,
        out_specs=[pl.BlockSpec((BLK, 12), lambda i: (i, 0)),
                   pl.BlockSpec((BLK, 32), lambda i: (i, 0)),
                   pl.BlockSpec((BLK, 72), lambda i: (i, 0))],
        out_shape=[jax.ShapeDtypeStruct((n, 12), jnp.float32),
                   jax.ShapeDtypeStruct((n, 32), jnp.float32),
                   jax.ShapeDtypeStruct((n, 72), jnp.float32)],
    )(e1, t1, *w1p, *u1p, *f2p)

    e2 = gather32(t2, idx).reshape(b_pad // 9, 9 * 32)

    wspecs = [_full_spec(x.shape) for x in (*w2p, *u2p, *f3p)]
    feas2, t3 = pl.pallas_call(
        _s2_kernel,
        grid=grid,
        in_specs=[pl.BlockSpec((BLK, 288), lambda i: (i, 0)),
                  pl.BlockSpec((BLK, 72), lambda i: (i, 0))] + wspecs,
        out_specs=[pl.BlockSpec((BLK, 24), lambda i: (i, 0)),
                   pl.BlockSpec((BLK, 48), lambda i: (i, 0))],
        out_shape=[jax.ShapeDtypeStruct((n, 24), jnp.float32),
                   jax.ShapeDtypeStruct((n, 48), jnp.float32)],
    )(e2, xyz, *w2p, *u2p, *f3p)

    e3 = gather48(t3, idx).reshape(b_pad // 9, 9 * 48)

    wspecs = [_full_spec(x.shape) for x in (*w3p, *u3p)]
    feas3 = pl.pallas_call(
        _s3_kernel,
        grid=grid,
        in_specs=[pl.BlockSpec((BLK, 432), lambda i: (i, 0)),
                  pl.BlockSpec((BLK, 72), lambda i: (i, 0))] + wspecs,
        out_specs=pl.BlockSpec((BLK, 48), lambda i: (i, 0)),
        out_shape=jax.ShapeDtypeStruct((n, 48), jnp.float32),
    )(e3, xyz, *w3p, *u3p)

    return jnp.concatenate([feas3, feas2, feas1, pf6], axis=1)


# speedup vs baseline: 6.5832x; 1.3608x over previous
import functools

import numpy as np

import jax
import jax.numpy as jnp
from jax import lax
from jax.experimental import pallas as pl
from jax.experimental.pallas import tpu as pltpu
from jax.experimental.pallas import tpu_sc as plsc

EPS_BN = 1e-5
BLK = 1000
NC, NS = 2, 16
NW = NC * NS
NBUF = 4


def _fold_pn(params):
    W1, b1, g1, be1, W2, b2, g2, be2 = params
    s = 1.0 / jnp.sqrt(1.0 + EPS_BN)
    A1 = W1.T * (g1 * s)[None, :]
    c1 = (b1 * (g1 * s) + be1).reshape(1, -1)
    A2 = W2.T * (g2 * s)[None, :]
    c2 = (b2 * (g2 * s) + be2).reshape(1, -1)
    return A1, c1, A2, c2


def _place(a, reps, rstride, cstride, roff, coff, shape):
    r, c = a.shape
    j = np.arange(reps)[:, None, None]
    rows = np.broadcast_to(j * rstride + np.arange(r)[None, :, None] + roff,
                           (reps, r, c)).reshape(-1)
    cols = np.broadcast_to(j * cstride + np.arange(c)[None, None, :] + coff,
                           (reps, r, c)).reshape(-1)
    return jnp.zeros(shape, jnp.float32).at[rows, cols].set(
        jnp.tile(a.reshape(-1), reps))


def _dot(x, y):
    return jnp.dot(x, y, preferred_element_type=jnp.float32)


def _pn2(x, a1, c1, a2, c2):
    return _dot(jnp.maximum(_dot(x, a1) + c1, 0.0), a2) + c2


def _colsumsq_kernel(x_ref, o_ref):
    p = jnp.sum(x_ref[...] * x_ref[...], axis=0, keepdims=True)

    @pl.when(pl.program_id(0) == 0)
    def _():
        o_ref[...] = p

    @pl.when(pl.program_id(0) > 0)
    def _():
        o_ref[...] += p


def _prep_kernel(ss_ref, x_ref, pn_ref, a1_ref, c1_ref, a2_ref, c2_ref,
                 t1_ref, nb_ref, pf6_ref):
    x = x_ref[...]
    inv = 1.0 / jnp.maximum(jnp.sqrt(ss_ref[0:1, 0:3]), 1e-12)
    pf6 = jnp.concatenate([x[:, 0:3] * inv, x[:, 3:6] * (1.0 / 255.0)], axis=1)
    f1 = _pn2(pf6, a1_ref[...], c1_ref[...], a2_ref[...], c2_ref[...])
    sel = jnp.concatenate([x[:, 0:3], x[:, 6:9]], axis=1)
    t1_ref[...] = jnp.concatenate(
        [sel, f1, jnp.zeros((BLK, 6), jnp.float32)], axis=1)
    pf6_ref[...] = pf6
    row = pl.program_id(0) * BLK + lax.broadcasted_iota(jnp.int32, (BLK, 9), 0)
    pnv = pn_ref[...]
    nb_ref[...] = jnp.where(pnv == 0, row, pnv)


def _make_gather(d, ch, b_pad):
    bpw = b_pad // NW
    n_ch = bpw // ch
    assert bpw % ch == 0 and ch % 8 == 0 and n_ch % NBUF == 0 and n_ch >= 2 * NBUF
    mesh = plsc.VectorSubcoreMesh(core_axis_name="c", subcore_axis_name="s")

    @functools.partial(
        pl.kernel,
        mesh=mesh,
        out_type=jax.ShapeDtypeStruct((b_pad, d), jnp.float32),
        scratch_types=[
            pltpu.VMEM((bpw,), jnp.int32),
            pltpu.VMEM((NBUF, ch, d), jnp.float32),
        ] + [pltpu.SemaphoreType.DMA] * (2 * NBUF),
        compiler_params=pltpu.CompilerParams(use_tc_tiling_on_sc=False),
    )
    def gather(table_hbm, idx_hbm, out_hbm, idx_v, bufs, *sems):
        wid = lax.axis_index("s") * NC + lax.axis_index("c")
        base = wid * bpw
        gsems, wsems = sems[:NBUF], sems[NBUF:]
        pltpu.sync_copy(idx_hbm.at[pl.ds(base, bpw)], idx_v)

        def g_copy(g, b):
            return pltpu.make_async_copy(
                table_hbm.at[idx_v.at[pl.ds(g * ch, ch)]], bufs.at[b], gsems[b])

        def w_copy(g, b):
            return pltpu.make_async_copy(
                bufs.at[b], out_hbm.at[pl.ds(base + g * ch, ch)], wsems[b])

        for b in range(NBUF):
            g_copy(b, b).start()

        def body(k, _):
            for b in range(NBUF):
                g = NBUF * k + b
                g_copy(g, b).wait()
                w_copy(g, b).start()
                w_copy(g, b).wait()
                g_copy(g + NBUF, b).start()
            return 0

        lax.fori_loop(0, n_ch // NBUF - 1, body, 0)
        for b in range(NBUF):
            g = n_ch - NBUF + b
            g_copy(g, b).wait()
            w_copy(g, b).start()
            w_copy(g, b).wait()

    return gather


def _s1_kernel(e_ref, t1_ref, a1bd, selb, c1t, a2emb, c2emb,
               u1p, uc1, u2a, uc2, fa1, fc1, fa2, fc2, p54, q54,
               feas_ref, t2_ref, xyz_ref):
    e = e_ref[...]
    sel6 = t1_ref[:, 0:6]
    h = jnp.maximum(_dot(e, a1bd[...]) + _dot(sel6, selb[...]) + c1t[...], 0.0)
    wemb = _dot(h, a2emb[...]) + c2emb[...]
    m = e * wemb
    feas = _pn2(m, u1p[...], uc1[...], u2a[...], uc2[...])
    feas_ref[...] = feas
    t2_ref[...] = _pn2(feas, fa1[...], fc1[...], fa2[...], fc2[...])
    xyz_ref[...] = _dot(e, p54[...]) + _dot(sel6, q54[...])


def _s23_kernel(e_ref, xyz_ref, a1bd, c1t, a2bd, c2t,
                u1a, uc1, u2a, uc2, *rest):
    has_next = len(rest) == 6
    e = e_ref[...]
    h = jnp.maximum(_dot(xyz_ref[...], a1bd[...]) + c1t[...], 0.0)
    w = _dot(h, a2bd[...]) + c2t[...]
    m = e * w
    feas = _pn2(m, u1a[...], uc1[...], u2a[...], uc2[...])
    if has_next:
        fa1, fc1, fa2, fc2, feas_ref, tn_ref = rest
        feas_ref[...] = feas
        tn_ref[...] = _pn2(feas, fa1[...], fc1[...], fa2[...], fc2[...])
    else:
        (feas_ref,) = rest
        feas_ref[...] = feas


def _full_spec(shape):
    return pl.BlockSpec(shape, lambda i: (0, 0))


def _row_spec(w):
    return pl.BlockSpec((BLK, w), lambda i: (i, 0))


def kernel(points_features, points_neighbor, p1f, p1w, p1u, p2f, p2w, p2u,
           p3f, p3w, p3u):
    n, m = points_features.shape[0], points_neighbor.shape[1]
    b = n * m
    ch24, ch48 = 720, 400
    b_pad = ((b + 8 * NW - 1) // (8 * NW)) * (8 * NW)
    while ((b_pad // NW) % (NBUF * ch24) or (b_pad // NW) % (NBUF * ch48)
           or b_pad % 9):
        b_pad += 8 * NW
    grid = (n // BLK,)

    f1p = _fold_pn(p1f)
    w1a1, w1c1, w1a2, w1c2 = _fold_pn(p1w)
    u1a1, u1c1, u1a2, u1c2 = _fold_pn(p1u)
    f2p = _fold_pn(p2f)
    w2a1, w2c1, w2a2, w2c2 = _fold_pn(p2w)
    u2a1, u2c1, u2a2, u2c2 = _fold_pn(p2u)
    f3p = _fold_pn(p3f)
    w3a1, w3c1, w3a2, w3c2 = _fold_pn(p3w)
    u3a1, u3c1, u3a2, u3c2 = _fold_pn(p3u)

    s1_a1bd = _place(w1a1, 9, 24, 12, 0, 0, (216, 108))
    s1_selb = jnp.tile(-w1a1, (1, 9))
    s1_c1t = jnp.tile(w1c1, (1, 9))
    s1_a2emb = _place(w1a2, 9, 12, 24, 0, 6, (108, 216))
    s1_c2emb = _place(w1c2, 9, 0, 24, 0, 6, (1, 216))
    ridx = 24 * (np.arange(108) // 12) + 6 + np.arange(108) % 12
    s1_u1p = jnp.zeros((216, 12), jnp.float32).at[ridx, :].set(u1a1)
    s1_p54 = _place(jnp.eye(6, dtype=jnp.float32), 9, 24, 6, 0, 0, (216, 54))
    s1_q54 = jnp.tile(-jnp.eye(6, dtype=jnp.float32), (1, 9))

    s2_a1bd = _place(w2a1, 9, 6, 24, 0, 0, (54, 216))
    s2_c1t = jnp.tile(w2c1, (1, 9))
    s2_a2bd = _place(w2a2, 9, 24, 24, 0, 0, (216, 216))
    s2_c2t = jnp.tile(w2c2, (1, 9))
    s3_a1bd = _place(w3a1, 9, 6, 48, 0, 0, (54, 432))
    s3_c1t = jnp.tile(w3c1, (1, 9))
    s3_a2bd = _place(w3a2, 9, 48, 48, 0, 0, (432, 432))
    s3_c2t = jnp.tile(w3c2, (1, 9))

    ss = pl.pallas_call(
        _colsumsq_kernel,
        grid=grid,
        in_specs=[_row_spec(9)],
        out_specs=pl.BlockSpec((1, 9), lambda i: (0, 0)),
        out_shape=jax.ShapeDtypeStruct((1, 9), jnp.float32),
    )(points_features)

    t1, nb, pf6 = pl.pallas_call(
        _prep_kernel,
        grid=grid,
        in_specs=[_full_spec((1, 9)), _row_spec(9), _row_spec(9)]
        + [_full_spec(x.shape) for x in f1p],
        out_specs=[_row_spec(24), _row_spec(9), _row_spec(6)],
        out_shape=[jax.ShapeDtypeStruct((n, 24), jnp.float32),
                   jax.ShapeDtypeStruct((n, 9), jnp.int32),
                   jax.ShapeDtypeStruct((n, 6), jnp.float32)],
    )(ss, points_features, points_neighbor, *f1p)

    idx = jnp.pad(nb.reshape(-1), (0, b_pad - b))

    gather24 = _make_gather(24, ch24, b_pad)
    gather48 = _make_gather(48, ch48, b_pad)

    e1 = gather24(t1, idx).reshape(b_pad // 9, 9 * 24)

    s1_w = (s1_a1bd, s1_selb, s1_c1t, s1_a2emb, s1_c2emb,
            s1_u1p, u1c1, u1a2, u1c2, *f2p, s1_p54, s1_q54)
    feas1, t2, xyz = pl.pallas_call(
        _s1_kernel,
        grid=grid,
        in_specs=[_row_spec(216), _row_spec(24)]
        + [_full_spec(x.shape) for x in s1_w],
        out_specs=[_row_spec(12), _row_spec(24), _row_spec(54)],
        out_shape=[jax.ShapeDtypeStruct((n, 12), jnp.float32),
                   jax.ShapeDtypeStruct((n, 24), jnp.float32),
                   jax.ShapeDtypeStruct((n, 54), jnp.float32)],
    )(e1, t1, *s1_w)

    e2 = gather24(t2, idx).reshape(b_pad // 9, 9 * 24)

    s2_w = (s2_a1bd, s2_c1t, s2_a2bd, s2_c2t, u2a1, u2c1, u2a2, u2c2, *f3p)
    feas2, t3 = pl.pallas_call(
        _s23_kernel,
        grid=grid,
        in_specs=[_row_spec(216), _row_spec(54)]
        + [_full_spec(x.shape) for x in s2_w],
        out_specs=[_row_spec(24), _row_spec(48)],
        out_shape=[jax.ShapeDtypeStruct((n, 24), jnp.float32),
                   jax.ShapeDtypeStruct((n, 48), jnp.float32)],
    )(e2, xyz, *s2_w)

    e3 = gather48(t3, idx).reshape(b_pad // 9, 9 * 48)

    s3_w = (s3_a1bd, s3_c1t, s3_a2bd, s3_c2t, u3a1, u3c1, u3a2, u3c2)
    feas3 = pl.pallas_call(
        _s23_kernel,
        grid=grid,
        in_specs=[_row_spec(432), _row_spec(54)]
        + [_full_spec(x.shape) for x in s3_w],
        out_specs=_row_spec(48),
        out_shape=jax.ShapeDtypeStruct((n, 48), jnp.float32),
    )(e3, xyz, *s3_w)

    return jnp.concatenate([feas3, feas2, feas1, pf6], axis=1)
